# speedup vs baseline: 2.8238x; 2.8238x over previous
import functools

import jax
import jax.numpy as jnp
from jax.experimental import pallas as pl
from jax.experimental.pallas import tpu as pltpu

N = 10000
D = 128
E = 320000
BLK = 1000
GRID = N // BLK
LN_EPS = 1e-5


def _ln_relu(u):
    mu = jnp.mean(u, axis=-1, keepdims=True)
    var = jnp.var(u, axis=-1, keepdims=True)
    return jax.nn.relu((u - mu) / jnp.sqrt(var + LN_EPS))


def _stage_a_body(x_ref, w_ref, deg_ref, hp_ref, dis_ref):
    dis = jax.lax.rsqrt(deg_ref[...])
    g = jnp.dot(x_ref[...], w_ref[...], preferred_element_type=jnp.float32)
    hp_ref[...] = g * dis
    dis_ref[...] = dis


def _stage_b_body(s_ref, hp_ref, dis_ref, b_ref, w2_ref, hp2_ref):
    u = dis_ref[...] * (s_ref[...] + hp_ref[...]) + b_ref[...]
    h = _ln_relu(u)
    g = jnp.dot(h, w2_ref[...], preferred_element_type=jnp.float32)
    hp2_ref[...] = g * dis_ref[...]


def _stage_c_body(s_ref, hp_ref, dis_ref, b_ref, o_ref):
    o_ref[...] = _ln_relu(dis_ref[...] * (s_ref[...] + hp_ref[...]) + b_ref[...])


_row_spec = pl.BlockSpec((BLK, D), lambda i: (i, 0))
_col1_spec = pl.BlockSpec((BLK, 1), lambda i: (i, 0))
_w_spec = pl.BlockSpec((D, D), lambda i: (0, 0))
_b_spec = pl.BlockSpec((1, D), lambda i: (0, 0))
_f32 = jnp.float32


def _stage_a(x, W, deg):
    return pl.pallas_call(
        _stage_a_body,
        grid=(GRID,),
        in_specs=[_row_spec, _w_spec, _col1_spec],
        out_specs=[_row_spec, _col1_spec],
        out_shape=[
            jax.ShapeDtypeStruct((N, D), _f32),
            jax.ShapeDtypeStruct((N, 1), _f32),
        ],
    )(x, W, deg)


def _stage_b(S, hp, dis, b, W2):
    return pl.pallas_call(
        _stage_b_body,
        grid=(GRID,),
        in_specs=[_row_spec, _row_spec, _col1_spec, _b_spec, _w_spec],
        out_specs=_row_spec,
        out_shape=jax.ShapeDtypeStruct((N, D), _f32),
    )(S, hp, dis, b, W2)


def _stage_c(S, hp, dis, b):
    return pl.pallas_call(
        _stage_c_body,
        grid=(GRID,),
        in_specs=[_row_spec, _row_spec, _col1_spec, _b_spec],
        out_specs=_row_spec,
        out_shape=jax.ShapeDtypeStruct((N, D), _f32),
    )(S, hp, dis, b)


def kernel(x, edge_index, edge_weight, W1, b1, W2, b2):
    row = edge_index[0].astype(jnp.int32)
    col = edge_index[1].astype(jnp.int32)
    w = edge_weight.astype(jnp.float32)

    deg = jax.ops.segment_sum(w, col, num_segments=N) + 1.0

    hp, dis = _stage_a(x, W1, deg.reshape(N, 1))
    S1 = jax.ops.segment_sum(hp[row] * w[:, None], col, num_segments=N)
    hp2 = _stage_b(S1, hp, dis, b1.reshape(1, D), W2)
    S2 = jax.ops.segment_sum(hp2[row] * w[:, None], col, num_segments=N)
    return _stage_c(S2, hp2, dis, b2.reshape(1, D))


# speedup vs baseline: 12.1603x; 4.3064x over previous
import functools

import jax
import jax.numpy as jnp
from jax import lax
from jax.experimental import pallas as pl
from jax.experimental.pallas import tpu as pltpu
from jax.experimental.pallas import tpu_sc as plsc

N = 10000
D = 128
E = 320000
LN_EPS = 1e-5

NW = 32
K = 128
CHUNKS = 79
EPW = CHUNKS * K
E_PAD = NW * EPW
ROWS_PER_TILE = N // 16

_f32 = jnp.float32
_i32 = jnp.int32

_sc_mesh = plsc.VectorSubcoreMesh(core_axis_name="c", subcore_axis_name="s")


@functools.partial(
    pl.kernel,
    out_type=jax.ShapeDtypeStruct((2 * N,), _f32),
    mesh=_sc_mesh,
    scratch_types=[
        pltpu.VMEM((CHUNKS, K), _i32),
        pltpu.VMEM((CHUNKS, K), _f32),
        pltpu.VMEM((N,), _f32),
        pltpu.VMEM_SHARED((N,), _f32),
    ],
)
def _deg_sc(col_hbm, w_hbm, dp_hbm, col_v, w_v, dvmem, dacc):
    cid = lax.axis_index("c")
    sid = lax.axis_index("s")
    wid = sid * 2 + cid

    pltpu.sync_copy(col_hbm.at[wid], col_v)
    pltpu.sync_copy(w_hbm.at[wid], w_v)

    @pl.when(sid == 0)
    def _zero():
        zero16 = jnp.zeros((16,), _f32)

        def zr(i, _):
            dvmem[pl.ds(i * 16, 16)] = zero16
            return 0

        lax.fori_loop(0, N // 16, zr, 0)
        pltpu.sync_copy(dvmem, dacc)

    plsc.subcore_barrier()

    def chunk(g, _):
        pltpu.sync_copy(w_v.at[g], dacc.at[col_v.at[g]], add=True)
        return 0

    lax.fori_loop(0, CHUNKS, chunk, 0)

    plsc.subcore_barrier()

    @pl.when(sid == 0)
    def _writeout():
        pltpu.sync_copy(dacc, dvmem)
        pltpu.sync_copy(dvmem, dp_hbm.at[pl.ds(cid * N, N)])


@functools.partial(
    pl.kernel,
    out_type=jax.ShapeDtypeStruct((2 * N, D), _f32),
    mesh=_sc_mesh,
    scratch_types=[
        pltpu.VMEM((CHUNKS, K), _i32),
        pltpu.VMEM((CHUNKS, K), _i32),
        pltpu.VMEM((CHUNKS, K), _f32),
        pltpu.VMEM((K, D), _f32),
        pltpu.VMEM_SHARED((N, D), _f32),
        pltpu.SemaphoreType.DMA,
    ],
)
def _msg_sc(hp_hbm, row_hbm, col_hbm, w_hbm, s_hbm,
            row_v, col_v, w_v, gbuf, acc, sem):
    cid = lax.axis_index("c")
    sid = lax.axis_index("s")
    wid = sid * 2 + cid

    pltpu.sync_copy(row_hbm.at[wid], row_v)
    pltpu.sync_copy(col_hbm.at[wid], col_v)
    pltpu.sync_copy(w_hbm.at[wid], w_v)

    zero16 = jnp.zeros((16,), _f32)

    def zr(i, _):
        for q in range(8):
            gbuf[i, pl.ds(q * 16, 16)] = zero16
        return 0

    lax.fori_loop(0, K, zr, 0)
    base = sid * 624
    for j in range(4):
        pltpu.sync_copy(gbuf, acc.at[pl.ds(base + j * 128, 128)])

    @pl.when(sid == 15)
    def _ztail_full():
        pltpu.sync_copy(gbuf, acc.at[pl.ds(base + 512, 128)])

    @pl.when(sid != 15)
    def _ztail_part():
        pltpu.sync_copy(gbuf.at[pl.ds(0, 112)], acc.at[pl.ds(base + 512, 112)])

    plsc.subcore_barrier()

    def chunk(g, _):
        pltpu.async_copy(hp_hbm.at[row_v.at[g]], gbuf, sem).wait()

        def grp(j, _):
            w16 = w_v[g, pl.ds(j * 16, 16)]
            for t in range(16):
                e = j * 16 + t
                sw = w16[t]
                for q in range(8):
                    sl = pl.ds(q * 16, 16)
                    gbuf[e, sl] = gbuf[e, sl] * sw
            return 0

        lax.fori_loop(0, 8, grp, 0)
        pltpu.sync_copy(gbuf, acc.at[col_v.at[g]], add=True)
        return 0

    lax.fori_loop(0, CHUNKS, chunk, 0)

    plsc.subcore_barrier()
    for j in range(4):
        pltpu.sync_copy(
            acc.at[pl.ds(base + j * 128, 128)],
            s_hbm.at[pl.ds(cid * N + base + j * 128, 128)],
        )

    @pl.when(sid == 15)
    def _wtail_full():
        pltpu.sync_copy(
            acc.at[pl.ds(base + 512, 128)],
            s_hbm.at[pl.ds(cid * N + base + 512, 128)],
        )

    @pl.when(sid != 15)
    def _wtail_part():
        pltpu.sync_copy(
            acc.at[pl.ds(base + 512, 112)],
            s_hbm.at[pl.ds(cid * N + base + 512, 112)],
        )


def _ln_relu(u):
    mu = jnp.mean(u, axis=-1, keepdims=True)
    var = jnp.var(u, axis=-1, keepdims=True)
    return jax.nn.relu((u - mu) / jnp.sqrt(var + LN_EPS))


def _stage_a_body(x_ref, w_ref, dp0_ref, dp1_ref, hp_ref, dis_ref):
    dis = jax.lax.rsqrt(dp0_ref[...] + dp1_ref[...] + 1.0)
    g = jnp.dot(x_ref[...], w_ref[...], preferred_element_type=_f32)
    hp_ref[...] = g * dis
    dis_ref[...] = dis


def _stage_b_body(s0_ref, s1_ref, hp_ref, dis_ref, b_ref, w2_ref, hp2_ref):
    u = dis_ref[...] * (s0_ref[...] + s1_ref[...] + hp_ref[...]) + b_ref[...]
    h = _ln_relu(u)
    g = jnp.dot(h, w2_ref[...], preferred_element_type=_f32)
    hp2_ref[...] = g * dis_ref[...]


def _stage_c_body(s0_ref, s1_ref, hp_ref, dis_ref, b_ref, o_ref):
    u = dis_ref[...] * (s0_ref[...] + s1_ref[...] + hp_ref[...]) + b_ref[...]
    o_ref[...] = _ln_relu(u)


BLK = 1000
GRID = N // BLK
_row_spec = pl.BlockSpec((BLK, D), lambda i: (i, 0))
_col1_spec = pl.BlockSpec((BLK, 1), lambda i: (i, 0))
_w_spec = pl.BlockSpec((D, D), lambda i: (0, 0))
_b_spec = pl.BlockSpec((1, D), lambda i: (0, 0))


def _stage_a(x, W, dp0, dp1):
    return pl.pallas_call(
        _stage_a_body,
        grid=(GRID,),
        in_specs=[_row_spec, _w_spec, _col1_spec, _col1_spec],
        out_specs=[_row_spec, _col1_spec],
        out_shape=[
            jax.ShapeDtypeStruct((N, D), _f32),
            jax.ShapeDtypeStruct((N, 1), _f32),
        ],
    )(x, W, dp0, dp1)


def _stage_b(S0, S1, hp, dis, b, W2):
    return pl.pallas_call(
        _stage_b_body,
        grid=(GRID,),
        in_specs=[_row_spec, _row_spec, _row_spec, _col1_spec, _b_spec, _w_spec],
        out_specs=_row_spec,
        out_shape=jax.ShapeDtypeStruct((N, D), _f32),
    )(S0, S1, hp, dis, b, W2)


def _stage_c(S0, S1, hp, dis, b):
    return pl.pallas_call(
        _stage_c_body,
        grid=(GRID,),
        in_specs=[_row_spec, _row_spec, _row_spec, _col1_spec, _b_spec],
        out_specs=_row_spec,
        out_shape=jax.ShapeDtypeStruct((N, D), _f32),
    )(S0, S1, hp, dis, b)


def kernel(x, edge_index, edge_weight, W1, b1, W2, b2):
    row = edge_index[0].astype(_i32)
    col = edge_index[1].astype(_i32)
    w = edge_weight.astype(_f32)

    pad = E_PAD - E
    row_r = jnp.concatenate([row, jnp.zeros((pad,), _i32)]).reshape(NW, CHUNKS, K)
    col_r = jnp.concatenate([col, jnp.zeros((pad,), _i32)]).reshape(NW, CHUNKS, K)
    w_r = jnp.concatenate([w, jnp.zeros((pad,), _f32)]).reshape(NW, CHUNKS, K)

    dp = _deg_sc(col_r, w_r)
    dp0 = dp[:N].reshape(N, 1)
    dp1 = dp[N:].reshape(N, 1)

    hp, dis = _stage_a(x, W1, dp0, dp1)
    S = _msg_sc(hp, row_r, col_r, w_r)
    hp2 = _stage_b(S[:N], S[N:], hp, dis, b1.reshape(1, D), W2)
    S2 = _msg_sc(hp2, row_r, col_r, w_r)
    return _stage_c(S2[:N], S2[N:], hp2, dis, b2.reshape(1, D))


# speedup vs baseline: 22.1860x; 1.8245x over previous
import functools

import jax
import jax.numpy as jnp
from jax import lax
from jax.experimental import pallas as pl
from jax.experimental.pallas import tpu as pltpu
from jax.experimental.pallas import tpu_sc as plsc

N = 10000
D = 128
DH = D // 2
E = 320000
LN_EPS = 1e-5

K = 112
CHUNKS = 179
EPW = CHUNKS * K
E_PAD = 16 * EPW

_f32 = jnp.float32
_i32 = jnp.int32

_sc_mesh = plsc.VectorSubcoreMesh(core_axis_name="c", subcore_axis_name="s")


@functools.partial(
    pl.kernel,
    out_type=jax.ShapeDtypeStruct((2 * N,), _f32),
    mesh=_sc_mesh,
    scratch_types=[
        pltpu.VMEM((CHUNKS, K), _i32),
        pltpu.VMEM((CHUNKS, K), _f32),
        pltpu.VMEM((N,), _f32),
        pltpu.VMEM_SHARED((N,), _f32),
    ],
)
def _deg_sc(col_hbm, w_hbm, dp_hbm, col_v, w_v, dvmem, dacc):
    cid = lax.axis_index("c")
    sid = lax.axis_index("s")

    pltpu.sync_copy(col_hbm.at[sid], col_v)
    pltpu.sync_copy(w_hbm.at[sid], w_v)

    @pl.when(sid == 0)
    def _zero():
        zero16 = jnp.zeros((16,), _f32)

        def zr(i, _):
            dvmem[pl.ds(i * 16, 16)] = zero16
            return 0

        lax.fori_loop(0, N // 16, zr, 0)
        pltpu.sync_copy(dvmem, dacc)

    plsc.subcore_barrier()

    lo = jnp.where(cid == 0, 0, 90)
    hi = jnp.where(cid == 0, 90, CHUNKS)

    def chunk(g, _):
        pltpu.sync_copy(w_v.at[g], dacc.at[col_v.at[g]], add=True)
        return 0

    lax.fori_loop(lo, hi, chunk, 0)

    plsc.subcore_barrier()

    @pl.when(sid == 0)
    def _writeout():
        pltpu.sync_copy(dacc, dvmem)
        pltpu.sync_copy(dvmem, dp_hbm.at[pl.ds(cid * N, N)])


@functools.partial(
    pl.kernel,
    out_type=jax.ShapeDtypeStruct((2, N, DH), _f32),
    mesh=_sc_mesh,
    scratch_types=[
        pltpu.VMEM((CHUNKS, K), _i32),
        pltpu.VMEM((CHUNKS, K), _i32),
        pltpu.VMEM((CHUNKS, K), _f32),
        pltpu.VMEM((K, DH), _f32),
        pltpu.VMEM((K, DH), _f32),
        pltpu.VMEM((K, DH), _f32),
        pltpu.VMEM((K, DH), _f32),
        pltpu.VMEM_SHARED((N, DH), _f32),
        pltpu.SemaphoreType.DMA,
        pltpu.SemaphoreType.DMA,
        pltpu.SemaphoreType.DMA,
        pltpu.SemaphoreType.DMA,
    ],
    compiler_params=pltpu.CompilerParams(use_tc_tiling_on_sc=False),
)
def _msg_sc(hp_hbm, row_hbm, col_hbm, w_hbm, s_hbm,
            row_v, col_v, w_v, g0, g1, s0, s1, acc,
            sem_g0, sem_g1, sem_s0, sem_s1):
    cid = lax.axis_index("c")
    sid = lax.axis_index("s")
    wid = cid * 16 + sid

    pltpu.sync_copy(row_hbm.at[wid], row_v)
    pltpu.sync_copy(col_hbm.at[sid], col_v)
    pltpu.sync_copy(w_hbm.at[sid], w_v)

    zero16 = jnp.zeros((16,), _f32)

    def zr(i, _):
        for q in range(DH // 16):
            s0[i, pl.ds(q * 16, 16)] = zero16
        return 0

    lax.fori_loop(0, K, zr, 0)
    base = sid * 624
    for j in range(5):
        pltpu.sync_copy(s0, acc.at[pl.ds(base + j * K, K)])

    @pl.when(sid == 15)
    def _ztail_full():
        pltpu.sync_copy(s0.at[pl.ds(0, 80)], acc.at[pl.ds(base + 5 * K, 80)])

    @pl.when(sid != 15)
    def _ztail_part():
        pltpu.sync_copy(s0.at[pl.ds(0, 64)], acc.at[pl.ds(base + 5 * K, 64)])

    plsc.subcore_barrier()

    def scale(gb, sb, g):
        def grp(j, _):
            w16 = w_v[g, pl.ds(j * 16, 16)]
            for t in range(16):
                e = j * 16 + t
                sw = w16[t]
                for q in range(DH // 16):
                    sl = pl.ds(q * 16, 16)
                    sb[e, sl] = gb[e, sl] * sw
            return 0

        lax.fori_loop(0, K // 16, grp, 0)

    pltpu.async_copy(hp_hbm.at[row_v.at[0]], g0, sem_g0)
    pltpu.async_copy(hp_hbm.at[row_v.at[1]], g1, sem_g1)

    def pipe(i, _):
        ga = 2 * i
        gb_ = 2 * i + 1

        pltpu.make_async_copy(hp_hbm.at[row_v.at[ga]], g0, sem_g0).wait()

        @pl.when(i >= 1)
        def _drain_a():
            pltpu.make_async_copy(s0, acc.at[col_v.at[ga - 2]], sem_s0).wait()

        scale(g0, s0, ga)
        pltpu.async_copy(s0, acc.at[col_v.at[ga]], sem_s0, add=True)
        pltpu.async_copy(hp_hbm.at[row_v.at[ga + 2]], g0, sem_g0)

        pltpu.make_async_copy(hp_hbm.at[row_v.at[gb_]], g1, sem_g1).wait()

        @pl.when(i >= 1)
        def _drain_b():
            pltpu.make_async_copy(s1, acc.at[col_v.at[gb_ - 2]], sem_s1).wait()

        scale(g1, s1, gb_)
        pltpu.async_copy(s1, acc.at[col_v.at[gb_]], sem_s1, add=True)

        @pl.when(i < (CHUNKS - 1) // 2 - 1)
        def _next_b():
            pltpu.async_copy(hp_hbm.at[row_v.at[gb_ + 2]], g1, sem_g1)

        return 0

    lax.fori_loop(0, (CHUNKS - 1) // 2, pipe, 0)

    last = CHUNKS - 1
    pltpu.make_async_copy(hp_hbm.at[row_v.at[last]], g0, sem_g0).wait()
    pltpu.make_async_copy(s0, acc.at[col_v.at[last - 2]], sem_s0).wait()
    scale(g0, s0, last)
    pltpu.async_copy(s0, acc.at[col_v.at[last]], sem_s0, add=True)
    pltpu.make_async_copy(s1, acc.at[col_v.at[last - 1]], sem_s1).wait()
    pltpu.make_async_copy(s0, acc.at[col_v.at[last]], sem_s0).wait()

    plsc.subcore_barrier()

    for j in range(5):
        pltpu.sync_copy(acc.at[pl.ds(base + j * K, K)],
                        s_hbm.at[cid, pl.ds(base + j * K, K)])

    @pl.when(sid == 15)
    def _wtail_full():
        pltpu.sync_copy(acc.at[pl.ds(base + 5 * K, 80)],
                        s_hbm.at[cid, pl.ds(base + 5 * K, 80)])

    @pl.when(sid != 15)
    def _wtail_part():
        pltpu.sync_copy(acc.at[pl.ds(base + 5 * K, 64)],
                        s_hbm.at[cid, pl.ds(base + 5 * K, 64)])


def _ln_relu(u):
    mu = jnp.mean(u, axis=-1, keepdims=True)
    var = jnp.var(u, axis=-1, keepdims=True)
    return jax.nn.relu((u - mu) / jnp.sqrt(var + LN_EPS))


def _stage_a_body(x_ref, w_ref, dp0_ref, dp1_ref, hp_ref, dis_ref):
    dis = jax.lax.rsqrt(dp0_ref[...] + dp1_ref[...] + 1.0)
    g = jnp.dot(x_ref[...], w_ref[...], preferred_element_type=_f32)
    hp_ref[...] = g * dis
    dis_ref[...] = dis


def _stage_b_body(s0_ref, s1_ref, hp_ref, dis_ref, b_ref, w2_ref, hp2_ref):
    s = jnp.concatenate([s0_ref[...], s1_ref[...]], axis=-1)
    u = dis_ref[...] * (s + hp_ref[...]) + b_ref[...]
    h = _ln_relu(u)
    g = jnp.dot(h, w2_ref[...], preferred_element_type=_f32)
    hp2_ref[...] = g * dis_ref[...]


def _stage_c_body(s0_ref, s1_ref, hp_ref, dis_ref, b_ref, o_ref):
    s = jnp.concatenate([s0_ref[...], s1_ref[...]], axis=-1)
    u = dis_ref[...] * (s + hp_ref[...]) + b_ref[...]
    o_ref[...] = _ln_relu(u)


BLK = 1000
GRID = N // BLK
_row_spec = pl.BlockSpec((BLK, D), lambda i: (i, 0))
_half_spec = pl.BlockSpec((BLK, DH), lambda i: (i, 0))
_col1_spec = pl.BlockSpec((BLK, 1), lambda i: (i, 0))
_w_spec = pl.BlockSpec((D, D), lambda i: (0, 0))
_b_spec = pl.BlockSpec((1, D), lambda i: (0, 0))


def _stage_a(x, W, dp0, dp1):
    return pl.pallas_call(
        _stage_a_body,
        grid=(GRID,),
        in_specs=[_row_spec, _w_spec, _col1_spec, _col1_spec],
        out_specs=[_row_spec, _col1_spec],
        out_shape=[
            jax.ShapeDtypeStruct((N, D), _f32),
            jax.ShapeDtypeStruct((N, 1), _f32),
        ],
    )(x, W, dp0, dp1)


def _stage_b(S0, S1, hp, dis, b, W2):
    return pl.pallas_call(
        _stage_b_body,
        grid=(GRID,),
        in_specs=[_half_spec, _half_spec, _row_spec, _col1_spec, _b_spec,
                  _w_spec],
        out_specs=_row_spec,
        out_shape=jax.ShapeDtypeStruct((N, D), _f32),
    )(S0, S1, hp, dis, b, W2)


def _stage_c(S0, S1, hp, dis, b):
    return pl.pallas_call(
        _stage_c_body,
        grid=(GRID,),
        in_specs=[_half_spec, _half_spec, _row_spec, _col1_spec, _b_spec],
        out_specs=_row_spec,
        out_shape=jax.ShapeDtypeStruct((N, D), _f32),
    )(S0, S1, hp, dis, b)


def kernel(x, edge_index, edge_weight, W1, b1, W2, b2):
    row = edge_index[0].astype(_i32)
    col = edge_index[1].astype(_i32)
    w = edge_weight.astype(_f32)

    pad = E_PAD - E
    rp = jnp.concatenate([row, jnp.zeros((pad,), _i32)]).reshape(16, CHUNKS, K)
    col_r = jnp.concatenate([col, jnp.zeros((pad,), _i32)]).reshape(16, CHUNKS, K)
    w_r = jnp.concatenate([w, jnp.zeros((pad,), _f32)]).reshape(16, CHUNKS, K)
    row2_r = jnp.concatenate([2 * rp, 2 * rp + 1]).reshape(32, CHUNKS, K)

    dp = _deg_sc(col_r, w_r)
    dp0 = dp[:N].reshape(N, 1)
    dp1 = dp[N:].reshape(N, 1)

    hp, dis = _stage_a(x, W1, dp0, dp1)
    S = _msg_sc(hp.reshape(2 * N, DH), row2_r, col_r, w_r)
    hp2 = _stage_b(S[0], S[1], hp, dis, b1.reshape(1, D), W2)
    S2 = _msg_sc(hp2.reshape(2 * N, DH), row2_r, col_r, w_r)
    return _stage_c(S2[0], S2[1], hp2, dis, b2.reshape(1, D))
